# Initial kernel scaffold; baseline (speedup 1.0000x reference)
import functools

import jax
import jax.numpy as jnp
from jax import lax
from jax.experimental import pallas as pl
from jax.experimental.pallas import tpu as pltpu
from jax.experimental.pallas import tpu_sc as plsc

N = 10000
E = 320000
D = 128
DH = D // 2
NC = 2
NS = 16
CH = 80
RPT = N // NS
ZR = 125
NPD = 10240
DRT = NPD // NS

_mesh = plsc.VectorSubcoreMesh(core_axis_name="c", subcore_axis_name="s")


def _fill_1d(ref, n, val, dtype):
    def body(i, _):
        ref[pl.ds(i * 16, 16)] = jnp.full((16,), val, dtype)
        return 0
    lax.fori_loop(0, n // 16, body, 0)


def _fill_2d(ref, rows, cols, val, dtype):
    def body(r, _):
        for j in range(cols // 16):
            ref[r, pl.ds(j * 16, 16)] = jnp.full((16,), val, dtype)
        return 0
    lax.fori_loop(0, rows, body, 0)


@functools.partial(
    pl.kernel,
    out_type=jax.ShapeDtypeStruct((NC, NPD), jnp.float32),
    mesh=_mesh,
    scratch_types=[
        pltpu.VMEM_SHARED((NPD,), jnp.float32),
        pltpu.VMEM((CH,), jnp.int32),
        pltpu.VMEM((CH,), jnp.float32),
        pltpu.VMEM((DRT,), jnp.float32),
    ],
)
def _deg_kernel(edge, out, acc, idx_buf, ones_buf, zbuf):
    c = lax.axis_index("c")
    s = lax.axis_index("s")
    _fill_1d(ones_buf, CH, 1.0, jnp.float32)
    _fill_1d(zbuf, DRT, 0.0, jnp.float32)
    zoff = pl.multiple_of(s * DRT, 8)
    pltpu.sync_copy(zbuf, acc.at[pl.ds(zoff, DRT)])
    plsc.subcore_barrier()
    ept = E // (NC * NS)
    tile_base = (c * NS + s) * ept

    def body(k, _):
        base = pl.multiple_of(tile_base + k * CH, 8)
        pltpu.sync_copy(edge.at[1, pl.ds(base, CH)], idx_buf)
        pltpu.sync_copy(ones_buf, acc.at[idx_buf], add=True)
        return 0

    lax.fori_loop(0, ept // CH, body, 0)
    plsc.subcore_barrier()
    pltpu.sync_copy(acc.at[pl.ds(zoff, DRT)], out.at[c, pl.ds(zoff, DRT)])


@functools.partial(
    pl.kernel,
    out_type=jax.ShapeDtypeStruct((NC, N, DH), jnp.float32),
    mesh=_mesh,
    scratch_types=[
        pltpu.VMEM_SHARED((N, DH), jnp.float32),
        pltpu.VMEM_SHARED((N, DH), jnp.float32),
        pltpu.VMEM((2, CH), jnp.int32),
        pltpu.VMEM((CH, DH), jnp.float32),
        pltpu.VMEM((ZR, DH), jnp.float32),
        pltpu.SemaphoreType.DMA,
    ],
)
def _agg_kernel(g, edge, out, sh_g, sh_acc, idx_buf, rows_buf, zbuf, sem):
    c = lax.axis_index("c")
    s = lax.axis_index("s")
    _fill_2d(zbuf, ZR, DH, 0.0, jnp.float32)
    row0 = s * RPT
    pltpu.sync_copy(g.at[c, pl.ds(row0, RPT), :], sh_g.at[pl.ds(row0, RPT), :])
    for z in range(RPT // ZR):
        pltpu.sync_copy(zbuf, sh_acc.at[pl.ds(row0 + z * ZR, ZR), :])
    plsc.subcore_barrier()

    ept = E // NS

    def body(k, _):
        base = pl.multiple_of(s * ept + k * CH, 8)
        pltpu.sync_copy(edge.at[:, pl.ds(base, CH)], idx_buf)
        pltpu.async_copy(sh_g.at[idx_buf.at[0]], rows_buf, sem).wait()
        pltpu.sync_copy(rows_buf, sh_acc.at[idx_buf.at[1]], add=True)
        return 0

    lax.fori_loop(0, ept // CH, body, 0)
    plsc.subcore_barrier()
    pltpu.sync_copy(sh_acc.at[pl.ds(row0, RPT), :], out.at[c, pl.ds(row0, RPT), :])


def _tc_first_body(x_ref, w_ref, degp_ref, g_ref, dinv_ref):
    deg = degp_ref[0] + degp_ref[1] + 1.0
    dinv = lax.rsqrt(deg)
    h = jnp.dot(x_ref[...], w_ref[...], preferred_element_type=jnp.float32)
    gv = h * dinv
    g_ref[0] = gv[:, :DH]
    g_ref[1] = gv[:, DH:]
    dinv_ref[...] = dinv


def _tc_mid_body(a_ref, g_ref, dinv_ref, w_ref, b_ref, og_ref):
    dinv = dinv_ref[...]
    f0 = jnp.maximum(dinv * (a_ref[0] + g_ref[0]) + b_ref[0:1, :DH], 0.0)
    f1 = jnp.maximum(dinv * (a_ref[1] + g_ref[1]) + b_ref[0:1, DH:], 0.0)
    h = (jnp.dot(f0, w_ref[:DH, :], preferred_element_type=jnp.float32)
         + jnp.dot(f1, w_ref[DH:, :], preferred_element_type=jnp.float32))
    gv = h * dinv
    og_ref[0] = gv[:, :DH]
    og_ref[1] = gv[:, DH:]


def _tc_last_body(a_ref, g_ref, dinv_ref, b_ref, o_ref):
    dinv = dinv_ref[...]
    o_ref[:, :DH] = dinv * (a_ref[0] + g_ref[0]) + b_ref[0:1, :DH]
    o_ref[:, DH:] = dinv * (a_ref[1] + g_ref[1]) + b_ref[0:1, DH:]


_tc_first = pl.pallas_call(
    _tc_first_body,
    out_shape=[
        jax.ShapeDtypeStruct((NC, N, DH), jnp.float32),
        jax.ShapeDtypeStruct((N, 1), jnp.float32),
    ],
)

_tc_mid = pl.pallas_call(
    _tc_mid_body,
    out_shape=jax.ShapeDtypeStruct((NC, N, DH), jnp.float32),
)

_tc_last = pl.pallas_call(
    _tc_last_body,
    out_shape=jax.ShapeDtypeStruct((N, D), jnp.float32),
)


def kernel(x, edge_index, W1, b1, W2, b2, W3, b3):
    degp = _deg_kernel(edge_index)[:, :N, None]
    g1, dinv = _tc_first(x, W1, degp)
    a1 = _agg_kernel(g1, edge_index)
    g2 = _tc_mid(a1, g1, dinv, W2, b1.reshape(1, D))
    a2 = _agg_kernel(g2, edge_index)
    g3 = _tc_mid(a2, g2, dinv, W3, b2.reshape(1, D))
    a3 = _agg_kernel(g3, edge_index)
    return _tc_last(a3, g3, dinv, b3.reshape(1, D))


# speedup vs baseline: 11.8269x; 11.8269x over previous
import functools

import jax
import jax.numpy as jnp
from jax import lax
from jax.experimental import pallas as pl
from jax.experimental.pallas import tpu as pltpu
from jax.experimental.pallas import tpu_sc as plsc

N = 10000
NP = 10240
E = 320000
D = 128
NC = 2
NS = 16
CH = 80
RPT = NP // NS
ZR = 128

_mesh = plsc.VectorSubcoreMesh(core_axis_name="c", subcore_axis_name="s")


def _fill_1d(ref, n, val, dtype):
    def body(i, _):
        ref[pl.ds(i * 16, 16)] = jnp.full((16,), val, dtype)
        return 0
    lax.fori_loop(0, n // 16, body, 0)


def _fill_2d(ref, rows, cols, val, dtype):
    def body(r, _):
        for j in range(cols // 16):
            ref[r, pl.ds(j * 16, 16)] = jnp.full((16,), val, dtype)
        return 0
    lax.fori_loop(0, rows, body, 0)


@functools.partial(
    pl.kernel,
    out_type=jax.ShapeDtypeStruct((NC, NP), jnp.float32),
    mesh=_mesh,
    scratch_types=[
        pltpu.VMEM_SHARED((NP,), jnp.float32),
        pltpu.VMEM((CH,), jnp.int32),
        pltpu.VMEM((CH,), jnp.float32),
        pltpu.VMEM((RPT,), jnp.float32),
    ],
)
def _deg_kernel(dst, out, acc, idx_buf, ones_buf, zbuf):
    c = lax.axis_index("c")
    s = lax.axis_index("s")
    _fill_1d(ones_buf, CH, 1.0, jnp.float32)
    _fill_1d(zbuf, RPT, 0.0, jnp.float32)
    zoff = pl.multiple_of(s * RPT, 8)
    pltpu.sync_copy(zbuf, acc.at[pl.ds(zoff, RPT)])
    plsc.subcore_barrier()
    ept = E // (NC * NS)
    tile_base = (c * NS + s) * ept

    def body(k, _):
        base = pl.multiple_of(tile_base + k * CH, 8)
        pltpu.sync_copy(dst.at[pl.ds(base, CH)], idx_buf)
        pltpu.sync_copy(ones_buf, acc.at[idx_buf], add=True)
        return 0

    lax.fori_loop(0, ept // CH, body, 0)
    plsc.subcore_barrier()
    pltpu.sync_copy(acc.at[pl.ds(zoff, RPT)], out.at[c, pl.ds(zoff, RPT)])


@functools.partial(
    pl.kernel,
    out_type=jax.ShapeDtypeStruct((NC, NP, D), jnp.float32),
    mesh=_mesh,
    scratch_types=[
        pltpu.VMEM_SHARED((NP, D), jnp.float32),
        pltpu.VMEM((CH,), jnp.int32),
        pltpu.VMEM((CH,), jnp.int32),
        pltpu.VMEM((CH, D), jnp.float32),
        pltpu.VMEM((ZR, D), jnp.float32),
        pltpu.SemaphoreType.DMA,
    ],
)
def _agg_kernel(g, src, dst, out, sh_acc, src_buf, dst_buf, rows_buf, zbuf, sem):
    c = lax.axis_index("c")
    s = lax.axis_index("s")
    _fill_2d(zbuf, ZR, D, 0.0, jnp.float32)
    row0 = s * RPT
    for z in range(RPT // ZR):
        pltpu.sync_copy(zbuf, sh_acc.at[pl.ds(row0 + z * ZR, ZR), :])
    plsc.subcore_barrier()

    ept = E // (NC * NS)
    tile_base = (c * NS + s) * ept

    def body(k, _):
        base = pl.multiple_of(tile_base + k * CH, 8)
        pltpu.sync_copy(src.at[pl.ds(base, CH)], src_buf)
        pltpu.sync_copy(dst.at[pl.ds(base, CH)], dst_buf)
        pltpu.async_copy(g.at[src_buf], rows_buf, sem).wait()
        pltpu.sync_copy(rows_buf, sh_acc.at[dst_buf], add=True)
        return 0

    lax.fori_loop(0, ept // CH, body, 0)
    plsc.subcore_barrier()
    pltpu.sync_copy(sh_acc.at[pl.ds(row0, RPT), :], out.at[c, pl.ds(row0, RPT), :])


def _tc_first_body(x_ref, w_ref, degp_ref, g_ref, dinv_ref):
    deg = degp_ref[0] + degp_ref[1] + 1.0
    dinv = lax.rsqrt(deg)
    h = jnp.dot(x_ref[...], w_ref[...], preferred_element_type=jnp.float32)
    g_ref[:N] = h * dinv[:N]
    g_ref[N:] = jnp.zeros((NP - N, D), jnp.float32)
    dinv_ref[...] = dinv


def _tc_mid_body(a_ref, g_ref, dinv_ref, w_ref, b_ref, og_ref):
    dinv = dinv_ref[...]
    agg = a_ref[0, :N] + a_ref[1, :N] + g_ref[:N]
    f = jnp.maximum(dinv[:N] * agg + b_ref[...], 0.0)
    h = jnp.dot(f, w_ref[...], preferred_element_type=jnp.float32)
    og_ref[:N] = h * dinv[:N]
    og_ref[N:] = jnp.zeros((NP - N, D), jnp.float32)


def _tc_last_body(a_ref, g_ref, dinv_ref, b_ref, o_ref):
    dinv = dinv_ref[...]
    agg = a_ref[0, :N] + a_ref[1, :N] + g_ref[:N]
    o_ref[...] = dinv[:N] * agg + b_ref[...]


_tc_first = pl.pallas_call(
    _tc_first_body,
    out_shape=[
        jax.ShapeDtypeStruct((NP, D), jnp.float32),
        jax.ShapeDtypeStruct((NP, 1), jnp.float32),
    ],
)

_tc_mid = pl.pallas_call(
    _tc_mid_body,
    out_shape=jax.ShapeDtypeStruct((NP, D), jnp.float32),
)

_tc_last = pl.pallas_call(
    _tc_last_body,
    out_shape=jax.ShapeDtypeStruct((N, D), jnp.float32),
)


def kernel(x, edge_index, W1, b1, W2, b2, W3, b3):
    src = edge_index[0]
    dst = edge_index[1]
    degp = _deg_kernel(dst)[:, :, None]
    g1, dinv = _tc_first(x, W1, degp)
    a1 = _agg_kernel(g1, src, dst)
    g2 = _tc_mid(a1, g1, dinv, W2, b1.reshape(1, D))
    a2 = _agg_kernel(g2, src, dst)
    g3 = _tc_mid(a2, g2, dinv, W3, b2.reshape(1, D))
    a3 = _agg_kernel(g3, src, dst)
    return _tc_last(a3, g3, dinv, b3.reshape(1, D))


# speedup vs baseline: 20.1196x; 1.7012x over previous
import functools

import jax
import jax.numpy as jnp
from jax import lax
from jax.experimental import pallas as pl
from jax.experimental.pallas import tpu as pltpu
from jax.experimental.pallas import tpu_sc as plsc

N = 10000
NP = 10240
E = 320000
EP = 327680
D = 128
NC = 2
NS = 16
CH = 80
NB = 4
EPT = EP // (NC * NS)
NCH = EPT // CH
RPT = NP // NS
ZR = 128

_mesh = plsc.VectorSubcoreMesh(core_axis_name="c", subcore_axis_name="s")


def _fill_1d(ref, n, val, dtype):
    def body(i, _):
        ref[pl.ds(i * 16, 16)] = jnp.full((16,), val, dtype)
        return 0
    lax.fori_loop(0, n // 16, body, 0)


def _fill_2d(ref, rows, cols, val, dtype):
    def body(r, _):
        for j in range(cols // 16):
            ref[r, pl.ds(j * 16, 16)] = jnp.full((16,), val, dtype)
        return 0
    lax.fori_loop(0, rows, body, 0)


@functools.partial(
    pl.kernel,
    out_type=jax.ShapeDtypeStruct((NC, NP), jnp.float32),
    mesh=_mesh,
    scratch_types=[
        pltpu.VMEM_SHARED((NP,), jnp.float32),
        pltpu.VMEM((CH,), jnp.int32),
        pltpu.VMEM((CH,), jnp.float32),
        pltpu.VMEM((RPT,), jnp.float32),
    ],
)
def _deg_kernel(dst, out, acc, idx_buf, ones_buf, zbuf):
    c = lax.axis_index("c")
    s = lax.axis_index("s")
    _fill_1d(ones_buf, CH, 1.0, jnp.float32)
    _fill_1d(zbuf, RPT, 0.0, jnp.float32)
    zoff = pl.multiple_of(s * RPT, 8)
    pltpu.sync_copy(zbuf, acc.at[pl.ds(zoff, RPT)])
    plsc.subcore_barrier()
    ept = EP // (NC * NS)
    tile_base = (c * NS + s) * ept

    def body(k, _):
        base = pl.multiple_of(tile_base + k * CH, 8)
        pltpu.sync_copy(dst.at[pl.ds(base, CH)], idx_buf)
        pltpu.sync_copy(ones_buf, acc.at[idx_buf], add=True)
        return 0

    lax.fori_loop(0, ept // CH, body, 0)
    plsc.subcore_barrier()
    pltpu.sync_copy(acc.at[pl.ds(zoff, RPT)], out.at[c, pl.ds(zoff, RPT)])


@functools.partial(
    pl.kernel,
    out_type=jax.ShapeDtypeStruct((NC, NP, D), jnp.float32),
    mesh=_mesh,
    scratch_types=[
        pltpu.VMEM_SHARED((NP, D), jnp.float32),
    ]
    + [pltpu.VMEM((CH,), jnp.int32)] * NB
    + [pltpu.VMEM((CH,), jnp.int32)] * NB
    + [pltpu.VMEM((CH, D), jnp.float32)] * NB
    + [pltpu.SemaphoreType.DMA] * (2 * NB),
)
def _agg_kernel(g, src, dst, out, sh_acc, *rest):
    src_bufs = rest[:NB]
    dst_bufs = rest[NB:2 * NB]
    rows_bufs = rest[2 * NB:3 * NB]
    isems = rest[3 * NB:3 * NB + NB]
    gsems = rest[3 * NB + NB:]
    c = lax.axis_index("c")
    s = lax.axis_index("s")
    _fill_2d(rows_bufs[0], CH, D, 0.0, jnp.float32)
    row0 = s * RPT
    for z in range(RPT // CH):
        pltpu.sync_copy(rows_bufs[0], sh_acc.at[pl.ds(row0 + z * CH, CH), :])
    plsc.subcore_barrier()

    tile_base = (c * NS + s) * EPT

    def outer(gi, _):
        base0 = pl.multiple_of(tile_base + gi * (NB * CH), 8)
        idx_d = []
        for b in range(NB):
            bb = pl.multiple_of(base0 + b * CH, 8)
            d1 = pltpu.async_copy(src.at[pl.ds(bb, CH)], src_bufs[b],
                                  isems[b])
            d2 = pltpu.async_copy(dst.at[pl.ds(bb, CH)], dst_bufs[b],
                                  isems[b])
            idx_d.append((d1, d2))
        g_d = []
        for b in range(NB):
            idx_d[b][0].wait()
            idx_d[b][1].wait()
            g_d.append(pltpu.async_copy(g.at[src_bufs[b]], rows_bufs[b],
                                        gsems[b]))
        for b in range(NB):
            g_d[b].wait()
            pltpu.sync_copy(rows_bufs[b], sh_acc.at[dst_bufs[b]],
                            add=True)
        return 0

    lax.fori_loop(0, NCH // NB, outer, 0)
    plsc.subcore_barrier()
    pltpu.sync_copy(sh_acc.at[pl.ds(row0, RPT), :], out.at[c, pl.ds(row0, RPT), :])


def _tc_first_body(x_ref, w_ref, degp_ref, g_ref, dinv_ref):
    deg = degp_ref[0] + degp_ref[1] + 1.0
    dinv = lax.rsqrt(deg)
    h = jnp.dot(x_ref[...], w_ref[...], preferred_element_type=jnp.float32)
    g_ref[:N] = h * dinv[:N]
    g_ref[N:] = jnp.zeros((NP - N, D), jnp.float32)
    dinv_ref[...] = dinv


def _tc_mid_body(a_ref, g_ref, dinv_ref, w_ref, b_ref, og_ref):
    dinv = dinv_ref[...]
    agg = a_ref[0, :N] + a_ref[1, :N] + g_ref[:N]
    f = jnp.maximum(dinv[:N] * agg + b_ref[...], 0.0)
    h = jnp.dot(f, w_ref[...], preferred_element_type=jnp.float32)
    og_ref[:N] = h * dinv[:N]
    og_ref[N:] = jnp.zeros((NP - N, D), jnp.float32)


def _tc_last_body(a_ref, g_ref, dinv_ref, b_ref, o_ref):
    dinv = dinv_ref[...]
    agg = a_ref[0, :N] + a_ref[1, :N] + g_ref[:N]
    o_ref[...] = dinv[:N] * agg + b_ref[...]


_tc_first = pl.pallas_call(
    _tc_first_body,
    out_shape=[
        jax.ShapeDtypeStruct((NP, D), jnp.float32),
        jax.ShapeDtypeStruct((NP, 1), jnp.float32),
    ],
)

_tc_mid = pl.pallas_call(
    _tc_mid_body,
    out_shape=jax.ShapeDtypeStruct((NP, D), jnp.float32),
)

_tc_last = pl.pallas_call(
    _tc_last_body,
    out_shape=jax.ShapeDtypeStruct((N, D), jnp.float32),
)


def kernel(x, edge_index, W1, b1, W2, b2, W3, b3):
    pad = N + (jnp.arange(EP - E, dtype=jnp.int32) % (NP - N))
    src = jnp.concatenate([edge_index[0], pad])
    dst = jnp.concatenate([edge_index[1], pad])
    degp = _deg_kernel(dst)[:, :, None]
    g1, dinv = _tc_first(x, W1, degp)
    a1 = _agg_kernel(g1, src, dst)
    g2 = _tc_mid(a1, g1, dinv, W2, b1.reshape(1, D))
    a2 = _agg_kernel(g2, src, dst)
    g3 = _tc_mid(a2, g2, dinv, W3, b2.reshape(1, D))
    a3 = _agg_kernel(g3, src, dst)
    return _tc_last(a3, g3, dinv, b3.reshape(1, D))


# speedup vs baseline: 28.6825x; 1.4256x over previous
import functools

import jax
import jax.numpy as jnp
from jax import lax
from jax.experimental import pallas as pl
from jax.experimental.pallas import tpu as pltpu
from jax.experimental.pallas import tpu_sc as plsc

N = 10000
NP = 10240
E = 320000
EP = 327680
D = 128
NC = 2
NS = 16
CH = 80
NB = 4
EPT = EP // (NC * NS)
NCH = EPT // CH
RPT = NP // NS
ZR = 128

_mesh = plsc.VectorSubcoreMesh(core_axis_name="c", subcore_axis_name="s")


def _fill_1d(ref, n, val, dtype):
    def body(i, _):
        ref[pl.ds(i * 16, 16)] = jnp.full((16,), val, dtype)
        return 0
    lax.fori_loop(0, n // 16, body, 0)


def _fill_2d(ref, rows, cols, val, dtype):
    def body(r, _):
        for j in range(cols // 16):
            ref[r, pl.ds(j * 16, 16)] = jnp.full((16,), val, dtype)
        return 0
    lax.fori_loop(0, rows, body, 0)


_DIB = 8
_DSB = 4


@functools.partial(
    pl.kernel,
    out_type=jax.ShapeDtypeStruct((NC, NP), jnp.float32),
    mesh=_mesh,
    scratch_types=[
        pltpu.VMEM_SHARED((NP,), jnp.float32),
        pltpu.VMEM((CH,), jnp.float32),
        pltpu.VMEM((RPT,), jnp.float32),
    ]
    + [pltpu.VMEM((CH,), jnp.int32)] * _DIB
    + [pltpu.SemaphoreType.DMA] * (_DIB + _DSB),
)
def _deg_kernel(dst, out, acc, ones_buf, zbuf, *rest):
    idx_bufs = rest[:_DIB]
    isems = rest[_DIB:2 * _DIB]
    ssems = rest[2 * _DIB:]
    c = lax.axis_index("c")
    s = lax.axis_index("s")
    _fill_1d(ones_buf, CH, 1.0, jnp.float32)
    _fill_1d(zbuf, RPT, 0.0, jnp.float32)
    zoff = pl.multiple_of(s * RPT, 8)
    pltpu.sync_copy(zbuf, acc.at[pl.ds(zoff, RPT)])
    plsc.subcore_barrier()
    nch = EPT // CH
    tile_base = (c * NS + s) * EPT

    def _fire_idx(k, jj):
        bb = pl.multiple_of(tile_base + k * CH, 8)
        pltpu.async_copy(dst.at[pl.ds(bb, CH)], idx_bufs[jj], isems[jj])

    def _drain_idx(jj):
        pltpu.make_async_copy(dst.at[pl.ds(0, CH)], idx_bufs[jj], isems[jj]).wait()

    def _drain_sc(r):
        pltpu.make_async_copy(out.at[0, pl.ds(0, CH)], ones_buf, ssems[r]).wait()

    for j in range(_DSB):
        _fire_idx(j, j)

    def outer(gi, _):
        k0 = gi * _DIB
        for jj in range(_DIB):
            k = k0 + jj
            r = jj % _DSB
            _drain_idx(jj)

            @pl.when(k >= _DSB)
            def _():
                _drain_sc(r)

            @pl.when(k + _DSB < nch)
            def _():
                _fire_idx(k + _DSB, (jj + _DSB) % _DIB)

            pltpu.async_copy(ones_buf, acc.at[idx_bufs[jj]], ssems[r], add=True)
        return 0

    lax.fori_loop(0, nch // _DIB, outer, 0)
    for r in range(_DSB):
        _drain_sc(r)
    plsc.subcore_barrier()
    pltpu.sync_copy(acc.at[pl.ds(zoff, RPT)], out.at[c, pl.ds(zoff, RPT)])


IB = 2 * NB
PF = NB


@functools.partial(
    pl.kernel,
    out_type=jax.ShapeDtypeStruct((NC, NP, D), jnp.float32),
    mesh=_mesh,
    scratch_types=[
        pltpu.VMEM_SHARED((NP, D), jnp.float32),
    ]
    + [pltpu.VMEM((CH,), jnp.int32)] * IB
    + [pltpu.VMEM((CH,), jnp.int32)] * IB
    + [pltpu.VMEM((CH, D), jnp.float32)] * NB
    + [pltpu.SemaphoreType.DMA] * (IB + 2 * NB),
)
def _agg_kernel(g, src, dst, out, sh_acc, *rest):
    src_bufs = rest[:IB]
    dst_bufs = rest[IB:2 * IB]
    rows_bufs = rest[2 * IB:2 * IB + NB]
    isems = rest[2 * IB + NB:2 * IB + NB + IB]
    gsems = rest[2 * IB + NB + IB:2 * IB + NB + IB + NB]
    ssems = rest[2 * IB + NB + IB + NB:]
    c = lax.axis_index("c")
    s = lax.axis_index("s")
    _fill_2d(rows_bufs[0], CH, D, 0.0, jnp.float32)
    row0 = s * RPT
    for z in range(RPT // CH):
        pltpu.sync_copy(rows_bufs[0], sh_acc.at[pl.ds(row0 + z * CH, CH), :])
    plsc.subcore_barrier()

    tile_base = (c * NS + s) * EPT

    def _fire_idx(k, jj):
        bb = pl.multiple_of(tile_base + k * CH, 8)
        pltpu.async_copy(src.at[pl.ds(bb, CH)], src_bufs[jj], isems[jj])
        pltpu.async_copy(dst.at[pl.ds(bb, CH)], dst_bufs[jj], isems[jj])

    def _drain_idx(jj):
        pltpu.make_async_copy(src.at[pl.ds(0, CH)], src_bufs[jj], isems[jj]).wait()
        pltpu.make_async_copy(dst.at[pl.ds(0, CH)], dst_bufs[jj], isems[jj]).wait()

    def _drain_rows(sems, r):
        pltpu.make_async_copy(g.at[pl.ds(0, CH), :], rows_bufs[r], sems[r]).wait()

    for j in range(PF):
        _fire_idx(j, j)

    def outer(gi, _):
        k0 = gi * IB
        for jj in range(IB):
            k = k0 + jj
            r = jj % NB
            _drain_idx(jj)

            @pl.when(k >= NB)
            def _():
                _drain_rows(ssems, r)

            @pl.when(k + PF < NCH)
            def _():
                _fire_idx(k + PF, (jj + PF) % IB)

            pltpu.async_copy(g.at[src_bufs[jj]], rows_bufs[r], gsems[r])

            @pl.when(k >= 1)
            def _():
                rp = (jj + NB - 1) % NB
                _drain_rows(gsems, rp)
                pltpu.async_copy(rows_bufs[rp],
                                 sh_acc.at[dst_bufs[(jj + IB - 1) % IB]],
                                 ssems[rp], add=True)
        return 0

    lax.fori_loop(0, NCH // IB, outer, 0)
    rl = (NCH - 1) % NB
    _drain_rows(gsems, rl)
    pltpu.async_copy(rows_bufs[rl], sh_acc.at[dst_bufs[(NCH - 1) % IB]],
                     ssems[rl], add=True)
    for r in range(NB):
        _drain_rows(ssems, r)
    plsc.subcore_barrier()
    pltpu.sync_copy(sh_acc.at[pl.ds(row0, RPT), :], out.at[c, pl.ds(row0, RPT), :])


def _tc_first_body(x_ref, w_ref, degp_ref, g_ref, dinv_ref):
    deg = degp_ref[0] + degp_ref[1] + 1.0
    dinv = lax.rsqrt(deg)
    h = jnp.dot(x_ref[...], w_ref[...], preferred_element_type=jnp.float32)
    g_ref[:N] = h * dinv[:N]
    g_ref[N:] = jnp.zeros((NP - N, D), jnp.float32)
    dinv_ref[...] = dinv


def _tc_mid_body(a_ref, g_ref, dinv_ref, w_ref, b_ref, og_ref):
    dinv = dinv_ref[...]
    agg = a_ref[0, :N] + a_ref[1, :N] + g_ref[:N]
    f = jnp.maximum(dinv[:N] * agg + b_ref[...], 0.0)
    h = jnp.dot(f, w_ref[...], preferred_element_type=jnp.float32)
    og_ref[:N] = h * dinv[:N]
    og_ref[N:] = jnp.zeros((NP - N, D), jnp.float32)


def _tc_last_body(a_ref, g_ref, dinv_ref, b_ref, o_ref):
    dinv = dinv_ref[...]
    agg = a_ref[0, :N] + a_ref[1, :N] + g_ref[:N]
    o_ref[...] = dinv[:N] * agg + b_ref[...]


_tc_first = pl.pallas_call(
    _tc_first_body,
    out_shape=[
        jax.ShapeDtypeStruct((NP, D), jnp.float32),
        jax.ShapeDtypeStruct((NP, 1), jnp.float32),
    ],
)

_tc_mid = pl.pallas_call(
    _tc_mid_body,
    out_shape=jax.ShapeDtypeStruct((NP, D), jnp.float32),
)

_tc_last = pl.pallas_call(
    _tc_last_body,
    out_shape=jax.ShapeDtypeStruct((N, D), jnp.float32),
)


def kernel(x, edge_index, W1, b1, W2, b2, W3, b3):
    pad = N + (jnp.arange(EP - E, dtype=jnp.int32) % (NP - N))
    src = jnp.concatenate([edge_index[0], pad])
    dst = jnp.concatenate([edge_index[1], pad])
    degp = _deg_kernel(dst)[:, :, None]
    g1, dinv = _tc_first(x, W1, degp)
    a1 = _agg_kernel(g1, src, dst)
    g2 = _tc_mid(a1, g1, dinv, W2, b1.reshape(1, D))
    a2 = _agg_kernel(g2, src, dst)
    g3 = _tc_mid(a2, g2, dinv, W3, b2.reshape(1, D))
    a3 = _agg_kernel(g3, src, dst)
    return _tc_last(a3, g3, dinv, b3.reshape(1, D))


# speedup vs baseline: 31.5524x; 1.1001x over previous
import functools

import jax
import jax.numpy as jnp
from jax import lax
from jax.experimental import pallas as pl
from jax.experimental.pallas import tpu as pltpu
from jax.experimental.pallas import tpu_sc as plsc

N = 10000
NP = 10240
E = 320000
EP = 327680
D = 128
NC = 2
NS = 16
CH = 80
NB = 4
EPT = EP // (NC * NS)
NCH = EPT // CH
RPT = NP // NS
ZR = 128

_mesh = plsc.VectorSubcoreMesh(core_axis_name="c", subcore_axis_name="s")


def _fill_1d(ref, n, val, dtype):
    def body(i, _):
        ref[pl.ds(i * 16, 16)] = jnp.full((16,), val, dtype)
        return 0
    lax.fori_loop(0, n // 16, body, 0)


def _fill_2d(ref, rows, cols, val, dtype):
    def body(r, _):
        for j in range(cols // 16):
            ref[r, pl.ds(j * 16, 16)] = jnp.full((16,), val, dtype)
        return 0
    lax.fori_loop(0, rows, body, 0)


_DIB = 8
_DSB = 4


@functools.partial(
    pl.kernel,
    out_type=jax.ShapeDtypeStruct((NC, NP), jnp.float32),
    mesh=_mesh,
    scratch_types=[
        pltpu.VMEM_SHARED((NP,), jnp.float32),
        pltpu.VMEM((CH,), jnp.float32),
        pltpu.VMEM((RPT,), jnp.float32),
    ]
    + [pltpu.VMEM((CH,), jnp.int32)] * _DIB
    + [pltpu.SemaphoreType.DMA] * (_DIB + _DSB),
)
def _deg_kernel(dst, out, acc, ones_buf, zbuf, *rest):
    idx_bufs = rest[:_DIB]
    isems = rest[_DIB:2 * _DIB]
    ssems = rest[2 * _DIB:]
    c = lax.axis_index("c")
    s = lax.axis_index("s")
    _fill_1d(ones_buf, CH, 1.0, jnp.float32)
    _fill_1d(zbuf, RPT, 0.0, jnp.float32)
    zoff = pl.multiple_of(s * RPT, 8)
    pltpu.sync_copy(zbuf, acc.at[pl.ds(zoff, RPT)])
    plsc.subcore_barrier()
    nch = EPT // CH
    tile_base = (c * NS + s) * EPT

    def _fire_idx(k, jj):
        bb = pl.multiple_of(tile_base + k * CH, 8)
        pltpu.async_copy(dst.at[pl.ds(bb, CH)], idx_bufs[jj], isems[jj])

    def _drain_idx(jj):
        pltpu.make_async_copy(dst.at[pl.ds(0, CH)], idx_bufs[jj], isems[jj]).wait()

    def _drain_sc(r):
        pltpu.make_async_copy(out.at[0, pl.ds(0, CH)], ones_buf, ssems[r]).wait()

    for j in range(_DSB):
        _fire_idx(j, j)

    def outer(gi, _):
        k0 = gi * _DIB
        for jj in range(_DIB):
            k = k0 + jj
            r = jj % _DSB
            _drain_idx(jj)

            @pl.when(k >= _DSB)
            def _():
                _drain_sc(r)

            @pl.when(k + _DSB < nch)
            def _():
                _fire_idx(k + _DSB, (jj + _DSB) % _DIB)

            pltpu.async_copy(ones_buf, acc.at[idx_bufs[jj]], ssems[r], add=True)
        return 0

    lax.fori_loop(0, nch // _DIB, outer, 0)
    for r in range(_DSB):
        _drain_sc(r)
    plsc.subcore_barrier()
    pltpu.sync_copy(acc.at[pl.ds(zoff, RPT)], out.at[c, pl.ds(zoff, RPT)])


IB = 2 * NB
PF = NB


@functools.partial(
    pl.kernel,
    out_type=jax.ShapeDtypeStruct((NC, NP, D), jnp.float32),
    mesh=_mesh,
    scratch_types=[
        pltpu.VMEM_SHARED((NP, D), jnp.float32),
    ]
    + [pltpu.VMEM((CH,), jnp.int32)] * IB
    + [pltpu.VMEM((CH,), jnp.int32)] * IB
    + [pltpu.VMEM((CH, D), jnp.float32)] * NB
    + [pltpu.SemaphoreType.DMA] * (IB + 2 * NB),
)
def _agg_kernel(g, src, dst, out, sh_acc, *rest):
    src_bufs = rest[:IB]
    dst_bufs = rest[IB:2 * IB]
    rows_bufs = rest[2 * IB:2 * IB + NB]
    isems = rest[2 * IB + NB:2 * IB + NB + IB]
    gsems = rest[2 * IB + NB + IB:2 * IB + NB + IB + NB]
    ssems = rest[2 * IB + NB + IB + NB:]
    c = lax.axis_index("c")
    s = lax.axis_index("s")
    _fill_2d(rows_bufs[0], CH, D, 0.0, jnp.float32)
    row0 = s * RPT
    for z in range(RPT // CH):
        pltpu.sync_copy(rows_bufs[0], sh_acc.at[pl.ds(row0 + z * CH, CH), :])
    plsc.subcore_barrier()

    tile_base = (c * NS + s) * EPT

    def _fire_idx(k, jj):
        bb = pl.multiple_of(tile_base + k * CH, 8)
        pltpu.async_copy(src.at[pl.ds(bb, CH)], src_bufs[jj], isems[jj])
        pltpu.async_copy(dst.at[pl.ds(bb, CH)], dst_bufs[jj], isems[jj])

    def _drain_idx(jj):
        pltpu.make_async_copy(src.at[pl.ds(0, CH)], src_bufs[jj], isems[jj]).wait()
        pltpu.make_async_copy(dst.at[pl.ds(0, CH)], dst_bufs[jj], isems[jj]).wait()

    def _drain_rows(sems, r):
        pltpu.make_async_copy(g.at[pl.ds(0, CH), :], rows_bufs[r], sems[r]).wait()

    for j in range(PF):
        _fire_idx(j, j)

    LAG = 2

    def outer(gi, _):
        k0 = gi * IB
        for jj in range(IB):
            k = k0 + jj
            r = jj % NB
            _drain_idx(jj)

            @pl.when(k >= NB)
            def _():
                _drain_rows(ssems, r)

            @pl.when(k + PF < NCH)
            def _():
                _fire_idx(k + PF, (jj + PF) % IB)

            pltpu.async_copy(g.at[src_bufs[jj]], rows_bufs[r], gsems[r])

            @pl.when(k >= LAG)
            def _():
                rp = (jj + NB - LAG) % NB
                _drain_rows(gsems, rp)
                pltpu.async_copy(rows_bufs[rp],
                                 sh_acc.at[dst_bufs[(jj + IB - LAG) % IB]],
                                 ssems[rp], add=True)
        return 0

    lax.fori_loop(0, NCH // IB, outer, 0)
    for k in range(NCH, NCH + LAG):
        rp = (k - LAG) % NB
        _drain_rows(gsems, rp)
        pltpu.async_copy(rows_bufs[rp], sh_acc.at[dst_bufs[(k - LAG) % IB]],
                         ssems[rp], add=True)
    for r in range(NB):
        _drain_rows(ssems, r)
    plsc.subcore_barrier()
    pltpu.sync_copy(sh_acc.at[pl.ds(row0, RPT), :], out.at[c, pl.ds(row0, RPT), :])


def _tc_first_body(x_ref, w_ref, degp_ref, g_ref, dinv_ref):
    deg = degp_ref[0] + degp_ref[1] + 1.0
    dinv = lax.rsqrt(deg)
    h = jnp.dot(x_ref[...], w_ref[...], preferred_element_type=jnp.float32)
    g_ref[:N] = h * dinv[:N]
    g_ref[N:] = jnp.zeros((NP - N, D), jnp.float32)
    dinv_ref[...] = dinv


def _tc_mid_body(a_ref, g_ref, dinv_ref, w_ref, b_ref, og_ref):
    dinv = dinv_ref[...]
    agg = a_ref[0, :N] + a_ref[1, :N] + g_ref[:N]
    f = jnp.maximum(dinv[:N] * agg + b_ref[...], 0.0)
    h = jnp.dot(f, w_ref[...], preferred_element_type=jnp.float32)
    og_ref[:N] = h * dinv[:N]
    og_ref[N:] = jnp.zeros((NP - N, D), jnp.float32)


def _tc_last_body(a_ref, g_ref, dinv_ref, b_ref, o_ref):
    dinv = dinv_ref[...]
    agg = a_ref[0, :N] + a_ref[1, :N] + g_ref[:N]
    o_ref[...] = dinv[:N] * agg + b_ref[...]


_tc_first = pl.pallas_call(
    _tc_first_body,
    out_shape=[
        jax.ShapeDtypeStruct((NP, D), jnp.float32),
        jax.ShapeDtypeStruct((NP, 1), jnp.float32),
    ],
)

_tc_mid = pl.pallas_call(
    _tc_mid_body,
    out_shape=jax.ShapeDtypeStruct((NP, D), jnp.float32),
)

_tc_last = pl.pallas_call(
    _tc_last_body,
    out_shape=jax.ShapeDtypeStruct((N, D), jnp.float32),
)


def kernel(x, edge_index, W1, b1, W2, b2, W3, b3):
    pad = N + (jnp.arange(EP - E, dtype=jnp.int32) % (NP - N))
    src = jnp.concatenate([edge_index[0], pad])
    dst = jnp.concatenate([edge_index[1], pad])
    degp = _deg_kernel(dst)[:, :, None]
    g1, dinv = _tc_first(x, W1, degp)
    a1 = _agg_kernel(g1, src, dst)
    g2 = _tc_mid(a1, g1, dinv, W2, b1.reshape(1, D))
    a2 = _agg_kernel(g2, src, dst)
    g3 = _tc_mid(a2, g2, dinv, W3, b2.reshape(1, D))
    a3 = _agg_kernel(g3, src, dst)
    return _tc_last(a3, g3, dinv, b3.reshape(1, D))


# speedup vs baseline: 34.1580x; 1.0826x over previous
import functools

import jax
import jax.numpy as jnp
from jax import lax
from jax.experimental import pallas as pl
from jax.experimental.pallas import tpu as pltpu
from jax.experimental.pallas import tpu_sc as plsc

N = 10000
NP = 10240
E = 320000
EP = 327680
D = 128
NC = 2
NS = 16
CH = 80
NB = 4
EPT = EP // (NC * NS)
NCH = EPT // CH
RPT = NP // NS
ZR = 128

_mesh = plsc.VectorSubcoreMesh(core_axis_name="c", subcore_axis_name="s")


def _fill_1d(ref, n, val, dtype):
    def body(i, _):
        ref[pl.ds(i * 16, 16)] = jnp.full((16,), val, dtype)
        return 0
    lax.fori_loop(0, n // 16, body, 0)


def _fill_2d(ref, rows, cols, val, dtype):
    def body(r, _):
        for j in range(cols // 16):
            ref[r, pl.ds(j * 16, 16)] = jnp.full((16,), val, dtype)
        return 0
    lax.fori_loop(0, rows, body, 0)


_DIB = 8
_DSB = 4


@functools.partial(
    pl.kernel,
    out_type=jax.ShapeDtypeStruct((NC, NP), jnp.float32),
    mesh=_mesh,
    scratch_types=[
        pltpu.VMEM_SHARED((NP,), jnp.float32),
        pltpu.VMEM((CH,), jnp.float32),
        pltpu.VMEM((RPT,), jnp.float32),
    ]
    + [pltpu.VMEM((CH,), jnp.int32)] * _DIB
    + [pltpu.SemaphoreType.DMA] * (_DIB + _DSB),
)
def _deg_kernel(dst, out, acc, ones_buf, zbuf, *rest):
    idx_bufs = rest[:_DIB]
    isems = rest[_DIB:2 * _DIB]
    ssems = rest[2 * _DIB:]
    c = lax.axis_index("c")
    s = lax.axis_index("s")
    _fill_1d(ones_buf, CH, 1.0, jnp.float32)
    _fill_1d(zbuf, RPT, 0.0, jnp.float32)
    zoff = pl.multiple_of(s * RPT, 8)
    pltpu.sync_copy(zbuf, acc.at[pl.ds(zoff, RPT)])
    plsc.subcore_barrier()
    nch = EPT // CH
    tile_base = (c * NS + s) * EPT

    def _fire_idx(k, jj):
        bb = pl.multiple_of(tile_base + k * CH, 8)
        pltpu.async_copy(dst.at[pl.ds(bb, CH)], idx_bufs[jj], isems[jj])

    def _drain_idx(jj):
        pltpu.make_async_copy(dst.at[pl.ds(0, CH)], idx_bufs[jj], isems[jj]).wait()

    def _drain_sc(r):
        pltpu.make_async_copy(out.at[0, pl.ds(0, CH)], ones_buf, ssems[r]).wait()

    for j in range(_DSB):
        _fire_idx(j, j)

    def outer(gi, _):
        k0 = gi * _DIB
        for jj in range(_DIB):
            k = k0 + jj
            r = jj % _DSB
            _drain_idx(jj)

            @pl.when(k >= _DSB)
            def _():
                _drain_sc(r)

            @pl.when(k + _DSB < nch)
            def _():
                _fire_idx(k + _DSB, (jj + _DSB) % _DIB)

            pltpu.async_copy(ones_buf, acc.at[idx_bufs[jj]], ssems[r], add=True)
        return 0

    lax.fori_loop(0, nch // _DIB, outer, 0)
    for r in range(_DSB):
        _drain_sc(r)
    plsc.subcore_barrier()
    pltpu.sync_copy(acc.at[pl.ds(zoff, RPT)], out.at[c, pl.ds(zoff, RPT)])


IB = 2 * NB
PF = NB


@functools.partial(
    pl.kernel,
    out_type=jax.ShapeDtypeStruct((NC, NP, D), jnp.float32),
    mesh=_mesh,
    scratch_types=[
        pltpu.VMEM_SHARED((NP, D), jnp.float32),
    ]
    + [pltpu.VMEM((CH,), jnp.int32)] * IB
    + [pltpu.VMEM((CH,), jnp.int32)] * IB
    + [pltpu.VMEM((CH, D), jnp.float32)] * NB
    + [pltpu.SemaphoreType.DMA] * (IB + 2 * NB),
)
def _agg_kernel(g, src, dst, out, sh_acc, *rest):
    src_bufs = rest[:IB]
    dst_bufs = rest[IB:2 * IB]
    rows_bufs = rest[2 * IB:2 * IB + NB]
    isems = rest[2 * IB + NB:2 * IB + NB + IB]
    gsems = rest[2 * IB + NB + IB:2 * IB + NB + IB + NB]
    ssems = rest[2 * IB + NB + IB + NB:]
    c = lax.axis_index("c")
    s = lax.axis_index("s")
    _fill_2d(rows_bufs[0], CH, D, 0.0, jnp.float32)
    row0 = s * RPT
    zds = [pltpu.async_copy(rows_bufs[0],
                            sh_acc.at[pl.ds(row0 + z * CH, CH), :],
                            ssems[z % NB]) for z in range(RPT // CH)]
    for d in zds:
        d.wait()
    plsc.subcore_barrier()

    tile_base = (c * NS + s) * EPT

    def _fire_idx(k, jj):
        bb = pl.multiple_of(tile_base + k * CH, 8)
        pltpu.async_copy(src.at[pl.ds(bb, CH)], src_bufs[jj], isems[jj])
        pltpu.async_copy(dst.at[pl.ds(bb, CH)], dst_bufs[jj], isems[jj])

    def _drain_idx(jj):
        pltpu.make_async_copy(src.at[pl.ds(0, CH)], src_bufs[jj], isems[jj]).wait()
        pltpu.make_async_copy(dst.at[pl.ds(0, CH)], dst_bufs[jj], isems[jj]).wait()

    def _drain_rows(sems, r):
        pltpu.make_async_copy(g.at[pl.ds(0, CH), :], rows_bufs[r], sems[r]).wait()

    for j in range(PF):
        _fire_idx(j, j)

    LAG = 3

    def outer(gi, _):
        k0 = gi * IB
        for jj in range(IB):
            k = k0 + jj
            r = jj % NB
            _drain_idx(jj)

            @pl.when(k >= NB)
            def _():
                _drain_rows(ssems, r)

            @pl.when(k + PF < NCH)
            def _():
                _fire_idx(k + PF, (jj + PF) % IB)

            pltpu.async_copy(g.at[src_bufs[jj]], rows_bufs[r], gsems[r])

            @pl.when(k >= LAG)
            def _():
                rp = (jj + NB - LAG) % NB
                _drain_rows(gsems, rp)
                pltpu.async_copy(rows_bufs[rp],
                                 sh_acc.at[dst_bufs[(jj + IB - LAG) % IB]],
                                 ssems[rp], add=True)
        return 0

    lax.fori_loop(0, NCH // IB, outer, 0)
    for k in range(NCH, NCH + LAG):
        rp = (k - LAG) % NB
        _drain_rows(gsems, rp)
        pltpu.async_copy(rows_bufs[rp], sh_acc.at[dst_bufs[(k - LAG) % IB]],
                         ssems[rp], add=True)
    for r in range(NB):
        _drain_rows(ssems, r)
    plsc.subcore_barrier()
    pltpu.sync_copy(sh_acc.at[pl.ds(row0, RPT), :], out.at[c, pl.ds(row0, RPT), :])


def _tc_first_body(x_ref, w_ref, degp_ref, g_ref, dinv_ref):
    deg = degp_ref[0] + degp_ref[1] + 1.0
    dinv = lax.rsqrt(deg)
    h = jnp.dot(x_ref[...], w_ref[...], preferred_element_type=jnp.float32)
    g_ref[:N] = h * dinv[:N]
    g_ref[N:] = jnp.zeros((NP - N, D), jnp.float32)
    dinv_ref[...] = dinv


def _tc_mid_body(a_ref, g_ref, dinv_ref, w_ref, b_ref, og_ref):
    dinv = dinv_ref[...]
    agg = a_ref[0, :N] + a_ref[1, :N] + g_ref[:N]
    f = jnp.maximum(dinv[:N] * agg + b_ref[...], 0.0)
    h = jnp.dot(f, w_ref[...], preferred_element_type=jnp.float32)
    og_ref[:N] = h * dinv[:N]
    og_ref[N:] = jnp.zeros((NP - N, D), jnp.float32)


def _tc_last_body(a_ref, g_ref, dinv_ref, b_ref, o_ref):
    dinv = dinv_ref[...]
    agg = a_ref[0, :N] + a_ref[1, :N] + g_ref[:N]
    o_ref[...] = dinv[:N] * agg + b_ref[...]


_tc_first = pl.pallas_call(
    _tc_first_body,
    out_shape=[
        jax.ShapeDtypeStruct((NP, D), jnp.float32),
        jax.ShapeDtypeStruct((NP, 1), jnp.float32),
    ],
)

_tc_mid = pl.pallas_call(
    _tc_mid_body,
    out_shape=jax.ShapeDtypeStruct((NP, D), jnp.float32),
)

_tc_last = pl.pallas_call(
    _tc_last_body,
    out_shape=jax.ShapeDtypeStruct((N, D), jnp.float32),
)


def kernel(x, edge_index, W1, b1, W2, b2, W3, b3):
    pad = N + (jnp.arange(EP - E, dtype=jnp.int32) % (NP - N))
    src = jnp.concatenate([edge_index[0], pad])
    dst = jnp.concatenate([edge_index[1], pad])
    degp = _deg_kernel(dst)[:, :, None]
    g1, dinv = _tc_first(x, W1, degp)
    a1 = _agg_kernel(g1, src, dst)
    g2 = _tc_mid(a1, g1, dinv, W2, b1.reshape(1, D))
    a2 = _agg_kernel(g2, src, dst)
    g3 = _tc_mid(a2, g2, dinv, W3, b2.reshape(1, D))
    a3 = _agg_kernel(g3, src, dst)
    return _tc_last(a3, g3, dinv, b3.reshape(1, D))
